# Initial kernel scaffold; baseline (speedup 1.0000x reference)
import jax
import jax.numpy as jnp
from jax.experimental import pallas as pl


def kernel(x, alpha, power, bias):
    raise NotImplementedError("write your pallas kernel here")


# speedup vs baseline: 16.8193x; 16.8193x over previous
import functools

import jax
import jax.numpy as jnp
import numpy as np
from jax.experimental import pallas as pl
from jax.experimental.pallas import tpu as pltpu

_SMOOTH = 0.015
_EPS = 1e-9
_W = 256
_ROWS = 512


def _scan_matrix(w: int) -> np.ndarray:
    k = np.arange(w, dtype=np.float64)[:, None]
    t = np.arange(w, dtype=np.float64)[None, :]
    a = np.where(t >= k, _SMOOTH * (1.0 - _SMOOTH) ** (t - k), 0.0)
    return a.astype(np.float32)


def _decay_vector(w: int) -> np.ndarray:
    t = np.arange(w, dtype=np.float64)
    return ((1.0 - _SMOOTH) ** (t + 1.0)).astype(np.float32)[None, :]


def _pcen_kernel(params_ref, x_ref, a_ref, d_ref, o_ref):
    neg_alpha = params_ref[0]
    power = params_ref[1]
    bias = params_ref[2]
    bias_pow = params_ref[3]

    a = a_ref[...]
    d = d_ref[...]
    t_total = x_ref.shape[1]
    n_chunks = t_total // _W

    carry = jnp.zeros((x_ref.shape[0], 1), jnp.float32)
    for c in range(n_chunks):
        xc = x_ref[:, c * _W:(c + 1) * _W]
        loc = jnp.dot(xc, a, preferred_element_type=jnp.float32,
                      precision=jax.lax.Precision.HIGHEST)
        m = loc + carry * d
        carry = m[:, _W - 1:_W]
        u = xc * jnp.exp(neg_alpha * jnp.log(_EPS + m)) + bias
        o_ref[:, c * _W:(c + 1) * _W] = jnp.exp(power * jnp.log(u)) - bias_pow


@functools.partial(jax.jit, static_argnames=())
def kernel(x, alpha, power, bias):
    b, f, t, ch = x.shape
    rows = b * f * ch
    x2 = x.reshape(rows, t)

    params = jnp.stack([
        -alpha[0],
        power[0],
        bias[0],
        jnp.power(bias[0], power[0]),
    ])

    a_mat = jnp.asarray(_scan_matrix(_W))
    d_vec = jnp.asarray(_decay_vector(_W))

    grid = (rows // _ROWS,)
    out = pl.pallas_call(
        _pcen_kernel,
        grid=grid,
        in_specs=[
            pl.BlockSpec(memory_space=pltpu.SMEM),
            pl.BlockSpec((_ROWS, t), lambda i: (i, 0)),
            pl.BlockSpec((_W, _W), lambda i: (0, 0)),
            pl.BlockSpec((1, _W), lambda i: (0, 0)),
        ],
        out_specs=pl.BlockSpec((_ROWS, t), lambda i: (i, 0)),
        out_shape=jax.ShapeDtypeStruct((rows, t), jnp.float32),
        compiler_params=pltpu.CompilerParams(
            dimension_semantics=("parallel",),
            vmem_limit_bytes=56 * 1024 * 1024,
        ),
    )(params, x2, a_mat, d_vec)
    return out.reshape(b, f, t, ch)


# speedup vs baseline: 20.8753x; 1.2411x over previous
import functools

import jax
import jax.numpy as jnp
import numpy as np
from jax.experimental import pallas as pl
from jax.experimental.pallas import tpu as pltpu

_SMOOTH = 0.015
_EPS = 1e-9
_W = 256
_ROWS = 512


def _scan_matrix(w: int) -> np.ndarray:
    k = np.arange(w, dtype=np.float64)[:, None]
    t = np.arange(w, dtype=np.float64)[None, :]
    a = np.where(t >= k, _SMOOTH * (1.0 - _SMOOTH) ** (t - k), 0.0)
    return a.astype(np.float32)


def _decay_vector(w: int) -> np.ndarray:
    t = np.arange(w, dtype=np.float64)
    return ((1.0 - _SMOOTH) ** (t + 1.0)).astype(np.float32)[None, :]


def _pcen_kernel(params_ref, x_ref, a_ref, d_ref, o_ref):
    neg_alpha = params_ref[0]
    power = params_ref[1]
    bias = params_ref[2]
    bias_pow = params_ref[3]

    a = a_ref[...]
    d = d_ref[...]
    t_total = x_ref.shape[1]
    n_chunks = t_total // _W

    carry = jnp.zeros((x_ref.shape[0], 1), jnp.float32)
    for c in range(n_chunks):
        xc = x_ref[:, c * _W:(c + 1) * _W]
        loc = jnp.dot(xc, a, preferred_element_type=jnp.float32,
                      precision=jax.lax.Precision.DEFAULT)
        m = loc + carry * d
        carry = m[:, _W - 1:_W]
        u = xc * jnp.exp(neg_alpha * jnp.log(_EPS + m)) + bias
        o_ref[:, c * _W:(c + 1) * _W] = jnp.exp(power * jnp.log(u)) - bias_pow


@functools.partial(jax.jit, static_argnames=())
def kernel(x, alpha, power, bias):
    b, f, t, ch = x.shape
    rows = b * f * ch
    x2 = x.reshape(rows, t)

    params = jnp.stack([
        -alpha[0],
        power[0],
        bias[0],
        jnp.power(bias[0], power[0]),
    ])

    a_mat = jnp.asarray(_scan_matrix(_W))
    d_vec = jnp.asarray(_decay_vector(_W))

    grid = (rows // _ROWS,)
    out = pl.pallas_call(
        _pcen_kernel,
        grid=grid,
        in_specs=[
            pl.BlockSpec(memory_space=pltpu.SMEM),
            pl.BlockSpec((_ROWS, t), lambda i: (i, 0)),
            pl.BlockSpec((_W, _W), lambda i: (0, 0)),
            pl.BlockSpec((1, _W), lambda i: (0, 0)),
        ],
        out_specs=pl.BlockSpec((_ROWS, t), lambda i: (i, 0)),
        out_shape=jax.ShapeDtypeStruct((rows, t), jnp.float32),
        compiler_params=pltpu.CompilerParams(
            dimension_semantics=("parallel",),
            vmem_limit_bytes=56 * 1024 * 1024,
        ),
    )(params, x2, a_mat, d_vec)
    return out.reshape(b, f, t, ch)


# speedup vs baseline: 44.1470x; 2.1148x over previous
import functools

import jax
import jax.numpy as jnp
import numpy as np
from jax.experimental import pallas as pl
from jax.experimental.pallas import tpu as pltpu

_SMOOTH = 1.5e-2
_EPS = 1e-9
_W = 128
_ROWS = 512


def _scan_matrices() -> np.ndarray:
    q = 1.0 - _SMOOTH
    k = np.arange(_W, dtype=np.float64)[:, None]
    t = np.arange(_W, dtype=np.float64)[None, :]
    a = np.where(t >= k, _SMOOTH * q ** (t - k), 0.0)
    abar = _SMOOTH * q ** (127.0 - k) * q ** (t + 1.0) / q ** 128.0
    return np.concatenate([a, abar], axis=1).astype(np.float32)


def _pcen_kernel(params_ref, x_ref, ad_ref, o_ref):
    neg_alpha = params_ref[0]
    power = params_ref[1]
    bias = params_ref[2]
    bias_pow = params_ref[3]

    r, c, w = x_ref.shape
    q = 1.0 - _SMOOTH
    w128 = q ** 128

    x3 = x_ref[...]
    lh = jnp.dot(x3.reshape(r * c, w), ad_ref[...],
                 preferred_element_type=jnp.float32)
    l3 = lh[:, :w].reshape(r, c, w)
    p = lh[:, w:].reshape(r, c, w)
    h3 = p
    for k in (1, 2, 4, 8):
        shifted = jnp.concatenate(
            [jnp.zeros((r, k, w), jnp.float32), p[:, :c - k, :]], axis=1)
        p = p + (w128 ** k) * shifted
    m = l3 + (p - h3)
    u = x3 * jnp.exp(neg_alpha * jnp.log(_EPS + m)) + bias
    o_ref[...] = jnp.exp(power * jnp.log(u)) - bias_pow


@functools.partial(jax.jit, static_argnames=())
def kernel(x, alpha, power, bias):
    b, f, t, ch = x.shape
    rows = b * f * ch
    n_chunks = t // _W
    x3 = x.reshape(rows, n_chunks, _W)

    params = jnp.stack([
        -alpha[0],
        power[0],
        bias[0],
        jnp.power(bias[0], power[0]),
    ])

    ad_mat = jnp.asarray(_scan_matrices())

    grid = (rows // _ROWS,)
    out = pl.pallas_call(
        _pcen_kernel,
        grid=grid,
        in_specs=[
            pl.BlockSpec(memory_space=pltpu.SMEM),
            pl.BlockSpec((_ROWS, n_chunks, _W), lambda i: (i, 0, 0)),
            pl.BlockSpec((_W, 2 * _W), lambda i: (0, 0)),
        ],
        out_specs=pl.BlockSpec((_ROWS, n_chunks, _W), lambda i: (i, 0, 0)),
        out_shape=jax.ShapeDtypeStruct((rows, n_chunks, _W), jnp.float32),
        compiler_params=pltpu.CompilerParams(
            dimension_semantics=("parallel",),
            vmem_limit_bytes=56 * 1024 * 1024,
        ),
    )(params, x3, ad_mat)
    return out.reshape(b, f, t, ch)


# speedup vs baseline: 45.1626x; 1.0230x over previous
import functools

import jax
import jax.numpy as jnp
import numpy as np
from jax.experimental import pallas as pl
from jax.experimental.pallas import tpu as pltpu

_SMOOTH = 1.5e-2
_EPS = 1e-9
_W = 128
_ROWS = 512


def _scan_matrices() -> np.ndarray:
    q = 1.0 - _SMOOTH
    k = np.arange(_W, dtype=np.float64)[:, None]
    t = np.arange(_W, dtype=np.float64)[None, :]
    a = np.where(t >= k, _SMOOTH * q ** (t - k), 0.0)
    abar = _SMOOTH * q ** (127.0 - k) * q ** (t + 1.0) / q ** 128.0
    return np.concatenate([a, abar], axis=1).astype(np.float32)


def _chunk_weights(c: int):
    ln_w = 128.0 * float(np.log1p(-_SMOOTH))
    i = jax.lax.broadcasted_iota(jnp.int32, (1, c, 1), 1).astype(jnp.float32)
    winv = jnp.exp(i * (-ln_w))
    wback = jnp.exp(i * ln_w)
    return winv, wback


def _compute(x3, ad, neg_alpha, power, bias, bias_pow):
    r, c, w = x3.shape
    winv, wback = _chunk_weights(c)
    lh = jnp.dot(x3.reshape(r * c, w), ad, preferred_element_type=jnp.float32)
    l3 = lh[:, :w].reshape(r, c, w)
    h3 = lh[:, w:].reshape(r, c, w)
    hp = h3 * winv
    p = hp
    for k in (1, 2, 4):
        shifted = jnp.concatenate(
            [jnp.zeros((r, k, w), jnp.float32), p[:, :c - k, :]], axis=1)
        p = p + shifted
    m = l3 + (p - hp) * wback
    u = x3 * jnp.exp(neg_alpha * jnp.log(_EPS + m)) + bias
    return jnp.exp(power * jnp.log(u)) - bias_pow


def _pcen_kernel(params_ref, x_hbm, ad_ref, o_hbm, xb, ob, in_sem, out_sem):
    i = pl.program_id(0)
    j = pl.program_id(1)
    nj = pl.num_programs(1)
    slot = jax.lax.rem(j, 2)
    nslot = 1 - slot

    def in_copy(jj, s):
        blk = i * nj + jj
        return pltpu.make_async_copy(
            x_hbm.at[pl.ds(blk * _ROWS, _ROWS)], xb.at[s], in_sem.at[s])

    def out_copy(jj, s):
        blk = i * nj + jj
        return pltpu.make_async_copy(
            ob.at[s], o_hbm.at[pl.ds(blk * _ROWS, _ROWS)], out_sem.at[s])

    @pl.when(j == 0)
    def _():
        in_copy(0, 0).start()

    @pl.when(j + 1 < nj)
    def _():
        in_copy(j + 1, nslot).start()

    in_copy(j, slot).wait()

    @pl.when(j >= 2)
    def _():
        out_copy(j - 2, slot).wait()

    sub = _ROWS // 2
    for g in range(2):
        ob[slot, g * sub:(g + 1) * sub] = _compute(
            xb[slot, g * sub:(g + 1) * sub], ad_ref[...], params_ref[0],
            params_ref[1], params_ref[2], params_ref[3])
    out_copy(j, slot).start()

    @pl.when(j == nj - 1)
    def _():
        out_copy(j - 1, nslot).wait()
        out_copy(j, slot).wait()


@functools.partial(jax.jit, static_argnames=())
def kernel(x, alpha, power, bias):
    b, f, t, ch = x.shape
    rows = b * f * ch
    n_chunks = t // _W
    x3 = x.reshape(rows, n_chunks, _W)

    params = jnp.stack([
        -alpha[0],
        power[0],
        bias[0],
        jnp.power(bias[0], power[0]),
    ])

    ad_mat = jnp.asarray(_scan_matrices())

    grid = (2, rows // _ROWS // 2)
    out = pl.pallas_call(
        _pcen_kernel,
        grid=grid,
        in_specs=[
            pl.BlockSpec(memory_space=pltpu.SMEM),
            pl.BlockSpec(memory_space=pl.ANY),
            pl.BlockSpec((_W, 2 * _W), lambda i, j: (0, 0)),
        ],
        out_specs=pl.BlockSpec(memory_space=pl.ANY),
        out_shape=jax.ShapeDtypeStruct((rows, n_chunks, _W), jnp.float32),
        scratch_shapes=[
            pltpu.VMEM((2, _ROWS, n_chunks, _W), jnp.float32),
            pltpu.VMEM((2, _ROWS, n_chunks, _W), jnp.float32),
            pltpu.SemaphoreType.DMA((2,)),
            pltpu.SemaphoreType.DMA((2,)),
        ],
        compiler_params=pltpu.CompilerParams(
            dimension_semantics=("parallel", "arbitrary"),
            vmem_limit_bytes=56 * 1024 * 1024,
        ),
    )(params, x3, ad_mat)
    return out.reshape(b, f, t, ch)
